# Initial kernel scaffold; baseline (speedup 1.0000x reference)
import jax
import jax.numpy as jnp
from jax.experimental import pallas as pl


def kernel(x, weights, paths):
    raise NotImplementedError("write your pallas kernel here")


# speedup vs baseline: 1.2441x; 1.2441x over previous
import jax
import jax.numpy as jnp
from jax import lax
from jax.experimental import pallas as pl
from jax.experimental.pallas import tpu as pltpu
from jax.experimental.pallas import tpu_sc as plsc

B = 16384
L = 32
NC = 2
NS = 16
NW = NC * NS
RPW = B // NW
IPW = RPW * L
GROUPS = RPW // 16


def _body(x_hbm, w_hbm, p_hbm, out_hbm, idx_v, gath_v, x_v, out_v, sem):
    wid = lax.axis_index("s") * NC + lax.axis_index("c")
    base = wid * RPW

    pltpu.sync_copy(p_hbm.at[pl.ds(base * L, IPW)], idx_v)
    pltpu.sync_copy(x_hbm.at[pl.ds(base, RPW)], x_v)
    pltpu.async_copy(w_hbm.at[idx_v], gath_v, sem).wait()

    lane = lax.broadcasted_iota(jnp.int32, (16,), 0) * L

    def g_body(g, carry):
        off = g * (16 * L)
        acc = x_v[pl.ds(pl.multiple_of(g * 16, 16), 16)]
        for l in range(L):
            acc = acc * plsc.load_gather(gath_v, [lane + (off + l)])
        out_v[pl.ds(pl.multiple_of(g * 16, 16), 16)] = acc
        return carry

    lax.fori_loop(0, GROUPS, g_body, 0)

    pltpu.sync_copy(out_v, out_hbm.at[pl.ds(base, RPW)])


def kernel(x, weights, paths):
    paths_flat = paths.reshape(-1).astype(jnp.int32)
    mesh = plsc.VectorSubcoreMesh(core_axis_name="c", subcore_axis_name="s")
    f = pl.kernel(
        _body,
        out_type=jax.ShapeDtypeStruct((B,), jnp.float32),
        mesh=mesh,
        scratch_types=[
            pltpu.VMEM((IPW,), jnp.int32),
            pltpu.VMEM((IPW,), jnp.float32),
            pltpu.VMEM((RPW,), jnp.float32),
            pltpu.VMEM((RPW,), jnp.float32),
            pltpu.SemaphoreType.DMA,
        ],
        compiler_params=pltpu.CompilerParams(needs_layout_passes=False),
    )
    return f(x, weights, paths_flat)


# speedup vs baseline: 1.2762x; 1.0258x over previous
import jax
import jax.numpy as jnp
from jax import lax
from jax.experimental import pallas as pl
from jax.experimental.pallas import tpu as pltpu
from jax.experimental.pallas import tpu_sc as plsc

B = 16384
L = 32
NC = 2
NS = 16
NW = NC * NS
RPW = B // NW
NCHUNK = 4
RC = RPW // NCHUNK
IC = RC * L


def _body(x_hbm, w_hbm, p_hbm, out_hbm,
          idx0, idx1, gath0, gath1, x_v, out_v, sem0, sem1):
    wid = lax.axis_index("s") * NC + lax.axis_index("c")
    base = wid * RPW

    idxs = [idx0, idx1]
    gaths = [gath0, gath1]
    sems = [sem0, sem1]
    copies = [None, None]

    pltpu.sync_copy(x_hbm.at[pl.ds(base, RPW)], x_v)
    pltpu.sync_copy(p_hbm.at[pl.ds(base * L, IC)], idx0)
    copies[0] = pltpu.async_copy(w_hbm.at[idx0], gath0, sem0)

    lane = lax.broadcasted_iota(jnp.int32, (16,), 0) * L

    for c in range(NCHUNK):
        cur = c % 2
        nxt = (c + 1) % 2
        if c + 1 < NCHUNK:
            pltpu.sync_copy(
                p_hbm.at[pl.ds(base * L + (c + 1) * IC, IC)], idxs[nxt])
            copies[nxt] = pltpu.async_copy(
                w_hbm.at[idxs[nxt]], gaths[nxt], sems[nxt])
        copies[cur].wait()
        gbuf = gaths[cur]

        def g_body(g, carry):
            off = g * (16 * L)
            row0 = pl.multiple_of(c * RC + g * 16, 16)
            acc = x_v[pl.ds(row0, 16)]
            for l in range(L):
                acc = acc * plsc.load_gather(gbuf, [lane + (off + l)])
            out_v[pl.ds(row0, 16)] = acc
            return carry

        lax.fori_loop(0, RC // 16, g_body, 0)

    pltpu.sync_copy(out_v, out_hbm.at[pl.ds(base, RPW)])


def kernel(x, weights, paths):
    paths_flat = paths.reshape(-1).astype(jnp.int32)
    mesh = plsc.VectorSubcoreMesh(core_axis_name="c", subcore_axis_name="s")
    f = pl.kernel(
        _body,
        out_type=jax.ShapeDtypeStruct((B,), jnp.float32),
        mesh=mesh,
        scratch_types=[
            pltpu.VMEM((IC,), jnp.int32),
            pltpu.VMEM((IC,), jnp.int32),
            pltpu.VMEM((IC,), jnp.float32),
            pltpu.VMEM((IC,), jnp.float32),
            pltpu.VMEM((RPW,), jnp.float32),
            pltpu.VMEM((RPW,), jnp.float32),
            pltpu.SemaphoreType.DMA,
            pltpu.SemaphoreType.DMA,
        ],
        compiler_params=pltpu.CompilerParams(needs_layout_passes=False),
    )
    return f(x, weights, paths_flat)


# speedup vs baseline: 1.3862x; 1.0862x over previous
import jax
import jax.numpy as jnp
from jax import lax
from jax.experimental import pallas as pl
from jax.experimental.pallas import tpu as pltpu
from jax.experimental.pallas import tpu_sc as plsc

B = 16384
L = 32
NC = 2
NS = 16
NW = NC * NS
RPW = B // NW
NCHUNK = 4
RC = RPW // NCHUNK
IC = RC * L


def _repack(p2d, idx1d):
    def r_body(r, carry):
        for cb in range(L // 16):
            idx1d[pl.ds(r * L + cb * 16, 16)] = p2d[r, pl.ds(cb * 16, 16)]
        return carry
    lax.fori_loop(0, RC, r_body, 0)


def _body(x_hbm, w_hbm, p_hbm, out_hbm,
          p2d0, p2d1, idx0, idx1, gath0, gath1, x_v, out_v, sem0, sem1):
    wid = lax.axis_index("s") * NC + lax.axis_index("c")
    base = wid * RPW

    p2ds = [p2d0, p2d1]
    idxs = [idx0, idx1]
    gaths = [gath0, gath1]
    sems = [sem0, sem1]
    copies = [None, None]

    pltpu.sync_copy(x_hbm.at[pl.ds(base, RPW)], x_v)
    pltpu.sync_copy(p_hbm.at[pl.ds(base, RC)], p2d0)
    _repack(p2d0, idx0)
    copies[0] = pltpu.async_copy(w_hbm.at[idx0], gath0, sem0)

    lane = lax.broadcasted_iota(jnp.int32, (16,), 0) * L

    for c in range(NCHUNK):
        cur = c % 2
        nxt = (c + 1) % 2
        if c + 1 < NCHUNK:
            pltpu.sync_copy(
                p_hbm.at[pl.ds(base + (c + 1) * RC, RC)], p2ds[nxt])
            _repack(p2ds[nxt], idxs[nxt])
            copies[nxt] = pltpu.async_copy(
                w_hbm.at[idxs[nxt]], gaths[nxt], sems[nxt])
        copies[cur].wait()
        gbuf = gaths[cur]

        def g_body(g, carry):
            off = g * (16 * L)
            row0 = pl.multiple_of(c * RC + g * 16, 16)
            acc = x_v[pl.ds(row0, 16)]
            for l in range(L):
                acc = acc * plsc.load_gather(gbuf, [lane + (off + l)])
            out_v[pl.ds(row0, 16)] = acc
            return carry

        lax.fori_loop(0, RC // 16, g_body, 0)

    pltpu.sync_copy(out_v, out_hbm.at[pl.ds(base, RPW)])


def kernel(x, weights, paths):
    paths32 = paths.astype(jnp.int32)
    mesh = plsc.VectorSubcoreMesh(core_axis_name="c", subcore_axis_name="s")
    f = pl.kernel(
        _body,
        out_type=jax.ShapeDtypeStruct((B,), jnp.float32),
        mesh=mesh,
        scratch_types=[
            pltpu.VMEM((RC, L), jnp.int32),
            pltpu.VMEM((RC, L), jnp.int32),
            pltpu.VMEM((IC,), jnp.int32),
            pltpu.VMEM((IC,), jnp.int32),
            pltpu.VMEM((IC,), jnp.float32),
            pltpu.VMEM((IC,), jnp.float32),
            pltpu.VMEM((RPW,), jnp.float32),
            pltpu.VMEM((RPW,), jnp.float32),
            pltpu.SemaphoreType.DMA,
            pltpu.SemaphoreType.DMA,
        ],
        compiler_params=pltpu.CompilerParams(needs_layout_passes=False),
    )
    return f(x, weights, paths32)
